# baseline (speedup 1.0000x reference)
import jax
import jax.numpy as jnp
from jax import lax
from jax.experimental import pallas as pl
from jax.experimental.pallas import tpu as pltpu

_K = 16
_D = 5


def _vq_body(cbt_ref, cb2_ref, csq_ref, zt_ref, zqt_ref, idx_ref,
             s1_ref, s2_ref, a1_ref, a2_ref):
    i = pl.program_id(0)
    n = pl.num_programs(0)
    zt = zt_ref[...]
    blk = zt.shape[1]

    crossT = jax.lax.dot_general(
        cb2_ref[...], zt, (((1,), (0,)), ((), ())),
        preferred_element_type=jnp.float32)
    scoreT = crossT + csq_ref[...]

    zz = zt * zt
    zsqT = jax.lax.dot_general(
        jnp.ones((1, _D), jnp.float32), zz, (((1,), (0,)), ((), ())),
        preferred_element_type=jnp.float32)

    mnT = jnp.min(scoreT, axis=0, keepdims=True)
    iiT = lax.broadcasted_iota(jnp.int32, scoreT.shape, 0)
    idxT = jnp.min(jnp.where(scoreT == mnT, iiT, _K), axis=0, keepdims=True)
    ohT = (iiT == idxT).astype(jnp.float32)

    codesT = jax.lax.dot_general(
        cbt_ref[...], ohT, (((1,), (0,)), ((), ())),
        preferred_element_type=jnp.float32)
    zqt_ref[...] = codesT
    idx_ref[...] = idxT.reshape(blk)

    min_d2 = jnp.maximum(mnT + zsqT, 0.0)

    @pl.when(i == 0)
    def _init():
        a1_ref[...] = jnp.zeros_like(a1_ref)
        a2_ref[...] = jnp.zeros_like(a2_ref)

    a1_ref[...] += min_d2
    a2_ref[...] += jnp.sqrt(min_d2)

    @pl.when(i == n - 1)
    def _fini():
        s1_ref[...] = jnp.sum(a1_ref[...], keepdims=True).reshape(1, 1)
        s2_ref[...] = jnp.sum(a2_ref[...], keepdims=True).reshape(1, 1)


@jax.jit
def kernel(neuromod_state, codebook):
    z = neuromod_state
    if z.ndim == 1:
        z = z[None, :]
    b, d = z.shape
    blk = 32768
    while b % blk != 0:
        blk //= 2
    grid = (b // blk,)

    zt = z.T
    cbt = codebook.T
    cb2 = -2.0 * codebook
    csq = jnp.sum(codebook * codebook, axis=1, keepdims=True)

    zqt, idx, s1, s2 = pl.pallas_call(
        _vq_body,
        grid=grid,
        in_specs=[
            pl.BlockSpec((d, _K), lambda i: (0, 0)),
            pl.BlockSpec((_K, d), lambda i: (0, 0)),
            pl.BlockSpec((_K, 1), lambda i: (0, 0)),
            pl.BlockSpec((d, blk), lambda i: (0, i)),
        ],
        out_specs=[
            pl.BlockSpec((d, blk), lambda i: (0, i)),
            pl.BlockSpec((blk,), lambda i: (i,)),
            pl.BlockSpec((1, 1), lambda i: (0, 0)),
            pl.BlockSpec((1, 1), lambda i: (0, 0)),
        ],
        out_shape=[
            jax.ShapeDtypeStruct((d, b), jnp.float32),
            jax.ShapeDtypeStruct((b,), jnp.int32),
            jax.ShapeDtypeStruct((1, 1), jnp.float32),
            jax.ShapeDtypeStruct((1, 1), jnp.float32),
        ],
        scratch_shapes=[
            pltpu.VMEM((1, blk), jnp.float32),
            pltpu.VMEM((1, blk), jnp.float32),
        ],
        compiler_params=pltpu.CompilerParams(
            dimension_semantics=("arbitrary",),
        ),
    )(cbt, cb2, csq, zt)

    commit_loss = (2.0 / (b * d)) * s1[0, 0]
    mean_dist = s2[0, 0] / b
    return zqt.T, idx, commit_loss, mean_dist


# speedup vs baseline: 1.0948x; 1.0948x over previous
import jax
import jax.numpy as jnp
from jax import lax
from jax.experimental import pallas as pl
from jax.experimental.pallas import tpu as pltpu

_K = 16
_D = 5


def _vq_body(cbt_ref, cb2_ref, csq_ref, zt_ref, zqt_ref, idx_ref,
             s1_ref, s2_ref, a1_ref, a2_ref):
    i = pl.program_id(0)
    n = pl.num_programs(0)
    zt = zt_ref[...]
    blk = zt.shape[1]

    crossT = jax.lax.dot_general(
        cb2_ref[...], zt, (((1,), (0,)), ((), ())),
        preferred_element_type=jnp.float32)
    scoreT = crossT + csq_ref[...]

    zz = zt * zt
    zsqT = jax.lax.dot_general(
        jnp.ones((1, _D), jnp.float32), zz, (((1,), (0,)), ((), ())),
        preferred_element_type=jnp.float32)

    mnT = jnp.min(scoreT, axis=0, keepdims=True)
    iiT = lax.broadcasted_iota(jnp.int32, scoreT.shape, 0)
    idxT = jnp.min(jnp.where(scoreT == mnT, iiT, _K), axis=0, keepdims=True)
    ohT = (iiT == idxT).astype(jnp.float32)

    codesT = jax.lax.dot_general(
        cbt_ref[...], ohT, (((1,), (0,)), ((), ())),
        preferred_element_type=jnp.float32)
    zqt_ref[...] = codesT
    idx_ref[...] = idxT.reshape(blk)

    min_d2 = jnp.maximum(mnT + zsqT, 0.0)

    @pl.when(i == 0)
    def _init():
        a1_ref[...] = jnp.zeros_like(a1_ref)
        a2_ref[...] = jnp.zeros_like(a2_ref)

    a1_ref[...] += min_d2
    a2_ref[...] += jnp.sqrt(min_d2)

    @pl.when(i == n - 1)
    def _fini():
        s1_ref[...] = jnp.sum(a1_ref[...], keepdims=True).reshape(1, 1)
        s2_ref[...] = jnp.sum(a2_ref[...], keepdims=True).reshape(1, 1)


@jax.jit
def kernel(neuromod_state, codebook):
    z = neuromod_state
    if z.ndim == 1:
        z = z[None, :]
    b, d = z.shape
    blk = 65536
    while b % blk != 0:
        blk //= 2
    grid = (b // blk,)

    zt = z.T
    cbt = codebook.T
    cb2 = -2.0 * codebook
    csq = jnp.sum(codebook * codebook, axis=1, keepdims=True)

    zqt, idx, s1, s2 = pl.pallas_call(
        _vq_body,
        grid=grid,
        in_specs=[
            pl.BlockSpec((d, _K), lambda i: (0, 0)),
            pl.BlockSpec((_K, d), lambda i: (0, 0)),
            pl.BlockSpec((_K, 1), lambda i: (0, 0)),
            pl.BlockSpec((d, blk), lambda i: (0, i)),
        ],
        out_specs=[
            pl.BlockSpec((d, blk), lambda i: (0, i)),
            pl.BlockSpec((blk,), lambda i: (i,)),
            pl.BlockSpec((1, 1), lambda i: (0, 0)),
            pl.BlockSpec((1, 1), lambda i: (0, 0)),
        ],
        out_shape=[
            jax.ShapeDtypeStruct((d, b), jnp.float32),
            jax.ShapeDtypeStruct((b,), jnp.int32),
            jax.ShapeDtypeStruct((1, 1), jnp.float32),
            jax.ShapeDtypeStruct((1, 1), jnp.float32),
        ],
        scratch_shapes=[
            pltpu.VMEM((1, blk), jnp.float32),
            pltpu.VMEM((1, blk), jnp.float32),
        ],
        compiler_params=pltpu.CompilerParams(
            dimension_semantics=("arbitrary",),
        ),
    )(cbt, cb2, csq, zt)

    commit_loss = (2.0 / (b * d)) * s1[0, 0]
    mean_dist = s2[0, 0] / b
    return zqt.T, idx, commit_loss, mean_dist
